# Initial kernel scaffold; baseline (speedup 1.0000x reference)
import jax
import jax.numpy as jnp
from jax.experimental import pallas as pl


def kernel(x_nchw, w1_oihw, g1, b1, w2_oihw, g2, b2):
    raise NotImplementedError("write your pallas kernel here")


# speedup vs baseline: 1.1347x; 1.1347x over previous
import functools

import jax
import jax.numpy as jnp
from jax.experimental import pallas as pl
from jax.experimental.pallas import tpu as pltpu

_EPS = 1e-5
_VMEM_LIMIT = 64 * 1024 * 1024


def _conv_taps(x, w_k, kw):
    L = x.shape[1]
    acc = jnp.dot(w_k[0], x, preferred_element_type=jnp.float32)
    for k in range(1, kw):
        tap = pltpu.roll(x, (L - k) % L, 1)
        acc = acc + jnp.dot(w_k[k], tap, preferred_element_type=jnp.float32)
    return acc


def _mask_w(acc, w_img, wo):
    lane = jax.lax.broadcasted_iota(jnp.int32, acc.shape, 1)
    return jnp.where(lane % w_img < wo, acc, 0.0)


def _p1_kernel(x_ref, w_ref, h_ref, s_ref, q_ref, *, kw, w_img, wo, nb):
    x = jnp.concatenate([x_ref[j] for j in range(nb)], axis=1)
    acc = _conv_taps(x, w_ref, kw)
    acc = _mask_w(acc, w_img, wo)
    cmid = acc.shape[0]
    h_ref[...] = acc.astype(jnp.bfloat16)
    s_ref[...] = jnp.sum(acc, axis=1, keepdims=True).reshape(1, cmid, 1)
    q_ref[...] = jnp.sum(acc * acc, axis=1, keepdims=True).reshape(1, cmid, 1)


def _p2_kernel(h_ref, scale_ref, shift_ref, w_ref, s_ref, q_ref,
               *, kw, w_img, wo):
    h = h_ref[...].astype(jnp.float32)
    y = jnp.maximum(h * scale_ref[...] + shift_ref[...], 0.0)
    acc = _conv_taps(y, w_ref, kw)
    acc = _mask_w(acc, w_img, wo)
    cout = acc.shape[0]
    s_ref[...] = jnp.sum(acc, axis=1, keepdims=True).reshape(1, cout, 1)
    q_ref[...] = jnp.sum(acc * acc, axis=1, keepdims=True).reshape(1, cout, 1)


def _p3_kernel(h_ref, scale1_ref, shift1_ref, w_ref, scale2_ref, shift2_ref,
               o_ref, *, kw, w_img, wo):
    h = h_ref[...].astype(jnp.float32)
    y = jnp.maximum(h * scale1_ref[...] + shift1_ref[...], 0.0)
    acc = _conv_taps(y, w_ref, kw)
    o_ref[...] = jnp.maximum(acc * scale2_ref[...] + shift2_ref[...], 0.0)


def _bn_scale_shift(part_sum, part_sumsq, count, gamma, beta):
    total = jnp.sum(part_sum, axis=(0, 2))
    total_sq = jnp.sum(part_sumsq, axis=(0, 2))
    mean = total / count
    var = jnp.maximum(total_sq / count - mean * mean, 0.0)
    inv = gamma * jax.lax.rsqrt(var + _EPS)
    scale = inv.reshape(-1, 1)
    shift = (beta - mean * inv).reshape(-1, 1)
    return scale, shift


@jax.jit
def _conv_block(x_nchw, w1_oihw, g1, b1, w2_oihw, g2, b2):
    N, Cin, H, W = x_nchw.shape
    Cmid = w1_oihw.shape[0]
    Cout = w2_oihw.shape[0]
    KW = w1_oihw.shape[3]
    Wo1 = W - (KW - 1)
    Wo2 = Wo1 - (KW - 1)
    M = N * H
    P = M * W
    HW = H * W

    NB = 8
    nblk1 = N // NB
    L1 = NB * HW

    x3 = x_nchw.reshape(N, Cin, HW)
    w1_k = jnp.transpose(w1_oihw[:, :, 0, :], (2, 0, 1))
    w2_k = jnp.transpose(w2_oihw[:, :, 0, :], (2, 0, 1))

    cparams = pltpu.CompilerParams(
        dimension_semantics=("parallel",),
        vmem_limit_bytes=_VMEM_LIMIT,
    )

    def stat_spec(nblk, c):
        return pl.BlockSpec((1, c, 1), lambda i: (i, 0, 0))

    def cvec_spec(c):
        return pl.BlockSpec((c, 1), lambda i: (0, 0))

    h1, s1, q1 = pl.pallas_call(
        functools.partial(_p1_kernel, kw=KW, w_img=W, wo=Wo1, nb=NB),
        grid=(nblk1,),
        in_specs=[
            pl.BlockSpec((NB, Cin, HW), lambda i: (i, 0, 0)),
            pl.BlockSpec((KW, Cmid, Cin), lambda i: (0, 0, 0)),
        ],
        out_specs=(
            pl.BlockSpec((Cmid, L1), lambda i: (0, i)),
            stat_spec(nblk1, Cmid),
            stat_spec(nblk1, Cmid),
        ),
        out_shape=(
            jax.ShapeDtypeStruct((Cmid, P), jnp.bfloat16),
            jax.ShapeDtypeStruct((nblk1, Cmid, 1), jnp.float32),
            jax.ShapeDtypeStruct((nblk1, Cmid, 1), jnp.float32),
        ),
        compiler_params=cparams,
    )(x3, w1_k)

    scale1, shift1 = _bn_scale_shift(s1, q1, float(M * Wo1), g1, b1)

    nblk2 = 16
    L2 = P // nblk2
    s2, q2 = pl.pallas_call(
        functools.partial(_p2_kernel, kw=KW, w_img=W, wo=Wo2),
        grid=(nblk2,),
        in_specs=[
            pl.BlockSpec((Cmid, L2), lambda i: (0, i)),
            cvec_spec(Cmid),
            cvec_spec(Cmid),
            pl.BlockSpec((KW, Cout, Cmid), lambda i: (0, 0, 0)),
        ],
        out_specs=(stat_spec(nblk2, Cout), stat_spec(nblk2, Cout)),
        out_shape=(
            jax.ShapeDtypeStruct((nblk2, Cout, 1), jnp.float32),
            jax.ShapeDtypeStruct((nblk2, Cout, 1), jnp.float32),
        ),
        compiler_params=cparams,
    )(h1, scale1, shift1, w2_k)

    scale2, shift2 = _bn_scale_shift(s2, q2, float(M * Wo2), g2, b2)

    nblk3 = 16
    L3 = P // nblk3
    h2 = pl.pallas_call(
        functools.partial(_p3_kernel, kw=KW, w_img=W, wo=Wo2),
        grid=(nblk3,),
        in_specs=[
            pl.BlockSpec((Cmid, L3), lambda i: (0, i)),
            cvec_spec(Cmid),
            cvec_spec(Cmid),
            pl.BlockSpec((KW, Cout, Cmid), lambda i: (0, 0, 0)),
            cvec_spec(Cout),
            cvec_spec(Cout),
        ],
        out_specs=pl.BlockSpec((Cout, L3), lambda i: (0, i)),
        out_shape=jax.ShapeDtypeStruct((Cout, P), jnp.float32),
        compiler_params=cparams,
    )(h1, scale1, shift1, w2_k, scale2, shift2)

    out = h2.reshape(Cout, N, H, W)[:, :, :, :Wo2]
    return jnp.transpose(out, (1, 0, 2, 3))


def kernel(x_nchw, w1_oihw, g1, b1, w2_oihw, g2, b2):
    return _conv_block(x_nchw, w1_oihw, g1, b1, w2_oihw, g2, b2)


# speedup vs baseline: 1.6717x; 1.4732x over previous
import functools

import jax
import jax.numpy as jnp
from jax.experimental import pallas as pl
from jax.experimental.pallas import tpu as pltpu

_EPS = 1e-5
_VMEM_LIMIT = 64 * 1024 * 1024


def _tap_stack(x, kw):
    L = x.shape[1]
    taps = [x]
    for k in range(1, kw):
        taps.append(pltpu.roll(x, (L - k) % L, 1))
    return jnp.concatenate(taps, axis=0)


def _mask_w(acc, w_img, wo):
    lane = jax.lax.broadcasted_iota(jnp.int32, acc.shape, 1)
    return jnp.where(lane % w_img < wo, acc, 0.0)


def _p1_kernel(x_ref, w_ref, h_ref, s_ref, q_ref, *, kw, w_img, wo, nb):
    x = jnp.concatenate([x_ref[j] for j in range(nb)], axis=1)
    acc = jnp.dot(w_ref[...], _tap_stack(x, kw),
                  preferred_element_type=jnp.float32)
    acc = _mask_w(acc, w_img, wo)
    cmid = acc.shape[0]
    h_ref[...] = acc.astype(jnp.bfloat16)
    s_ref[...] = jnp.sum(acc, axis=1, keepdims=True).reshape(1, cmid, 1)
    q_ref[...] = jnp.sum(acc * acc, axis=1, keepdims=True).reshape(1, cmid, 1)


def _p2_kernel(h_ref, scale_ref, shift_ref, w_ref, s_ref, q_ref,
               *, kw, w_img, wo):
    h = h_ref[...].astype(jnp.float32)
    y = jnp.maximum(h * scale_ref[...] + shift_ref[...], 0.0)
    acc = jnp.dot(w_ref[...], _tap_stack(y, kw),
                  preferred_element_type=jnp.float32)
    acc = _mask_w(acc, w_img, wo)
    cout = acc.shape[0]
    s_ref[...] = jnp.sum(acc, axis=1, keepdims=True).reshape(1, cout, 1)
    q_ref[...] = jnp.sum(acc * acc, axis=1, keepdims=True).reshape(1, cout, 1)


def _p3_kernel(h_ref, scale1_ref, shift1_ref, w_ref, scale2_ref, shift2_ref,
               o_ref, *, kw, w_img, wo, nb, h_img):
    h = h_ref[...].astype(jnp.float32)
    y = jnp.maximum(h * scale1_ref[...] + shift1_ref[...], 0.0)
    acc = jnp.dot(w_ref[...], _tap_stack(y, kw),
                  preferred_element_type=jnp.float32)
    z = jnp.maximum(acc * scale2_ref[...] + shift2_ref[...], 0.0)
    cout = z.shape[0]
    z3 = z.reshape(cout, nb * h_img, w_img)
    for j in range(nb):
        o_ref[j] = z3[:, j * h_img:(j + 1) * h_img, :wo]


def _bn_scale_shift(part_sum, part_sumsq, count, gamma, beta):
    total = jnp.sum(part_sum, axis=(0, 2))
    total_sq = jnp.sum(part_sumsq, axis=(0, 2))
    mean = total / count
    var = jnp.maximum(total_sq / count - mean * mean, 0.0)
    inv = gamma * jax.lax.rsqrt(var + _EPS)
    scale = inv.reshape(-1, 1)
    shift = (beta - mean * inv).reshape(-1, 1)
    return scale, shift


@jax.jit
def _conv_block(x_nchw, w1_oihw, g1, b1, w2_oihw, g2, b2):
    N, Cin, H, W = x_nchw.shape
    Cmid = w1_oihw.shape[0]
    Cout = w2_oihw.shape[0]
    KW = w1_oihw.shape[3]
    Wo1 = W - (KW - 1)
    Wo2 = Wo1 - (KW - 1)
    M = N * H
    P = M * W
    HW = H * W

    NB = 8
    nblk1 = N // NB
    L1 = NB * HW

    x3 = x_nchw.reshape(N, Cin, HW)
    w1_cat = jnp.transpose(w1_oihw[:, :, 0, :], (0, 2, 1)).reshape(Cmid, KW * Cin)
    w2_cat = jnp.transpose(w2_oihw[:, :, 0, :], (0, 2, 1)).reshape(Cout, KW * Cmid)

    cparams = pltpu.CompilerParams(
        dimension_semantics=("parallel",),
        vmem_limit_bytes=_VMEM_LIMIT,
    )

    def stat_spec(c):
        return pl.BlockSpec((1, c, 1), lambda i: (i, 0, 0))

    def cvec_spec(c):
        return pl.BlockSpec((c, 1), lambda i: (0, 0))

    def wmat_spec(o, k):
        return pl.BlockSpec((o, k), lambda i: (0, 0))

    h1, s1, q1 = pl.pallas_call(
        functools.partial(_p1_kernel, kw=KW, w_img=W, wo=Wo1, nb=NB),
        grid=(nblk1,),
        in_specs=[
            pl.BlockSpec((NB, Cin, HW), lambda i: (i, 0, 0)),
            wmat_spec(Cmid, KW * Cin),
        ],
        out_specs=(
            pl.BlockSpec((Cmid, L1), lambda i: (0, i)),
            stat_spec(Cmid),
            stat_spec(Cmid),
        ),
        out_shape=(
            jax.ShapeDtypeStruct((Cmid, P), jnp.bfloat16),
            jax.ShapeDtypeStruct((nblk1, Cmid, 1), jnp.float32),
            jax.ShapeDtypeStruct((nblk1, Cmid, 1), jnp.float32),
        ),
        compiler_params=cparams,
    )(x3, w1_cat)

    scale1, shift1 = _bn_scale_shift(s1, q1, float(M * Wo1), g1, b1)

    nblk2 = 16
    L2 = P // nblk2
    s2, q2 = pl.pallas_call(
        functools.partial(_p2_kernel, kw=KW, w_img=W, wo=Wo2),
        grid=(nblk2,),
        in_specs=[
            pl.BlockSpec((Cmid, L2), lambda i: (0, i)),
            cvec_spec(Cmid),
            cvec_spec(Cmid),
            wmat_spec(Cout, KW * Cmid),
        ],
        out_specs=(stat_spec(Cout), stat_spec(Cout)),
        out_shape=(
            jax.ShapeDtypeStruct((nblk2, Cout, 1), jnp.float32),
            jax.ShapeDtypeStruct((nblk2, Cout, 1), jnp.float32),
        ),
        compiler_params=cparams,
    )(h1, scale1, shift1, w2_cat)

    scale2, shift2 = _bn_scale_shift(s2, q2, float(M * Wo2), g2, b2)

    nblk3 = N // NB
    L3 = NB * HW
    out = pl.pallas_call(
        functools.partial(_p3_kernel, kw=KW, w_img=W, wo=Wo2, nb=NB, h_img=H),
        grid=(nblk3,),
        in_specs=[
            pl.BlockSpec((Cmid, L3), lambda i: (0, i)),
            cvec_spec(Cmid),
            cvec_spec(Cmid),
            wmat_spec(Cout, KW * Cmid),
            cvec_spec(Cout),
            cvec_spec(Cout),
        ],
        out_specs=pl.BlockSpec((NB, Cout, H, Wo2), lambda i: (i, 0, 0, 0)),
        out_shape=jax.ShapeDtypeStruct((N, Cout, H, Wo2), jnp.float32),
        compiler_params=cparams,
    )(h1, scale1, shift1, w2_cat, scale2, shift2)

    return out


def kernel(x_nchw, w1_oihw, g1, b1, w2_oihw, g2, b2):
    return _conv_block(x_nchw, w1_oihw, g1, b1, w2_oihw, g2, b2)


# speedup vs baseline: 4.6609x; 2.7881x over previous
import functools

import jax
import jax.numpy as jnp
from jax.experimental import pallas as pl
from jax.experimental.pallas import tpu as pltpu

_EPS = 1e-5
_VMEM_LIMIT = 64 * 1024 * 1024


def _tap_stack(x, kw, stride):
    L = x.shape[1]
    taps = [x]
    for k in range(1, kw):
        taps.append(pltpu.roll(x, (L - k * stride) % L, 1))
    return jnp.concatenate(taps, axis=0)


def _mask_w(acc, w_img, wo, n):
    lane = jax.lax.broadcasted_iota(jnp.int32, acc.shape, 1)
    return jnp.where((lane // n) % w_img < wo, acc, 0.0)


def _p1_kernel(x_ref, w_ref, h_ref, s_ref, q_ref, *, kw, w_img, wo, n):
    cin = x_ref.shape[0]
    x = x_ref[...].reshape(cin, -1)
    acc = jnp.dot(w_ref[...], _tap_stack(x, kw, n),
                  preferred_element_type=jnp.float32)
    acc = _mask_w(acc, w_img, wo, n)
    cmid = acc.shape[0]
    h_ref[...] = acc.astype(jnp.bfloat16)
    s_ref[...] = jnp.sum(acc, axis=1, keepdims=True).reshape(1, cmid, 1)
    q_ref[...] = jnp.sum(acc * acc, axis=1, keepdims=True).reshape(1, cmid, 1)


def _p2_kernel(h_ref, scale_ref, shift_ref, w_ref, s_ref, q_ref,
               *, kw, w_img, wo, n):
    h = h_ref[...].astype(jnp.float32)
    y = jnp.maximum(h * scale_ref[...] + shift_ref[...], 0.0)
    acc = jnp.dot(w_ref[...], _tap_stack(y, kw, n),
                  preferred_element_type=jnp.float32)
    acc = _mask_w(acc, w_img, wo, n)
    cout = acc.shape[0]
    s_ref[...] = jnp.sum(acc, axis=1, keepdims=True).reshape(1, cout, 1)
    q_ref[...] = jnp.sum(acc * acc, axis=1, keepdims=True).reshape(1, cout, 1)


def _p3_kernel(h_ref, scale1_ref, shift1_ref, w_ref, scale2_ref, shift2_ref,
               o_ref, *, kw, w_img, wo, n, hb):
    h = h_ref[...].astype(jnp.float32)
    y = jnp.maximum(h * scale1_ref[...] + shift1_ref[...], 0.0)
    acc = jnp.dot(w_ref[...], _tap_stack(y, kw, n),
                  preferred_element_type=jnp.float32)
    z = jnp.maximum(acc * scale2_ref[...] + shift2_ref[...], 0.0)
    cout = z.shape[0]
    z4 = z.reshape(cout, hb, w_img, n)
    o_ref[...] = jnp.transpose(z4, (0, 2, 1, 3))[:, :wo]


def _bn_scale_shift(part_sum, part_sumsq, count, gamma, beta):
    total = jnp.sum(part_sum, axis=(0, 2))
    total_sq = jnp.sum(part_sumsq, axis=(0, 2))
    mean = total / count
    var = jnp.maximum(total_sq / count - mean * mean, 0.0)
    inv = gamma * jax.lax.rsqrt(var + _EPS)
    scale = inv.reshape(-1, 1)
    shift = (beta - mean * inv).reshape(-1, 1)
    return scale, shift


@jax.jit
def _conv_block(x_nchw, w1_oihw, g1, b1, w2_oihw, g2, b2):
    N, Cin, H, W = x_nchw.shape
    Cmid = w1_oihw.shape[0]
    Cout = w2_oihw.shape[0]
    KW = w1_oihw.shape[3]
    Wo1 = W - (KW - 1)
    Wo2 = Wo1 - (KW - 1)
    P = N * H * W

    HB = 8
    nblk = H // HB
    L = HB * W * N

    x_t = jnp.transpose(x_nchw, (1, 2, 3, 0))
    w1_cat = jnp.transpose(w1_oihw[:, :, 0, :], (0, 2, 1)).reshape(Cmid, KW * Cin)
    w2_cat = jnp.transpose(w2_oihw[:, :, 0, :], (0, 2, 1)).reshape(Cout, KW * Cmid)

    cparams = pltpu.CompilerParams(
        dimension_semantics=("parallel",),
        vmem_limit_bytes=_VMEM_LIMIT,
    )

    def stat_spec(c):
        return pl.BlockSpec((1, c, 1), lambda i: (i, 0, 0))

    def cvec_spec(c):
        return pl.BlockSpec((c, 1), lambda i: (0, 0))

    def wmat_spec(o, k):
        return pl.BlockSpec((o, k), lambda i: (0, 0))

    def lane_spec(c):
        return pl.BlockSpec((c, L), lambda i: (0, i))

    h1, s1, q1 = pl.pallas_call(
        functools.partial(_p1_kernel, kw=KW, w_img=W, wo=Wo1, n=N),
        grid=(nblk,),
        in_specs=[
            pl.BlockSpec((Cin, HB, W, N), lambda i: (0, i, 0, 0)),
            wmat_spec(Cmid, KW * Cin),
        ],
        out_specs=(
            lane_spec(Cmid),
            stat_spec(Cmid),
            stat_spec(Cmid),
        ),
        out_shape=(
            jax.ShapeDtypeStruct((Cmid, P), jnp.bfloat16),
            jax.ShapeDtypeStruct((nblk, Cmid, 1), jnp.float32),
            jax.ShapeDtypeStruct((nblk, Cmid, 1), jnp.float32),
        ),
        compiler_params=cparams,
    )(x_t, w1_cat)

    scale1, shift1 = _bn_scale_shift(s1, q1, float(N * H * Wo1), g1, b1)

    s2, q2 = pl.pallas_call(
        functools.partial(_p2_kernel, kw=KW, w_img=W, wo=Wo2, n=N),
        grid=(nblk,),
        in_specs=[
            lane_spec(Cmid),
            cvec_spec(Cmid),
            cvec_spec(Cmid),
            wmat_spec(Cout, KW * Cmid),
        ],
        out_specs=(stat_spec(Cout), stat_spec(Cout)),
        out_shape=(
            jax.ShapeDtypeStruct((nblk, Cout, 1), jnp.float32),
            jax.ShapeDtypeStruct((nblk, Cout, 1), jnp.float32),
        ),
        compiler_params=cparams,
    )(h1, scale1, shift1, w2_cat)

    scale2, shift2 = _bn_scale_shift(s2, q2, float(N * H * Wo2), g2, b2)

    out_t = pl.pallas_call(
        functools.partial(_p3_kernel, kw=KW, w_img=W, wo=Wo2, n=N, hb=HB),
        grid=(nblk,),
        in_specs=[
            lane_spec(Cmid),
            cvec_spec(Cmid),
            cvec_spec(Cmid),
            wmat_spec(Cout, KW * Cmid),
            cvec_spec(Cout),
            cvec_spec(Cout),
        ],
        out_specs=pl.BlockSpec((Cout, Wo2, HB, N), lambda i: (0, 0, i, 0)),
        out_shape=jax.ShapeDtypeStruct((Cout, Wo2, H, N), jnp.float32),
        compiler_params=cparams,
    )(h1, scale1, shift1, w2_cat, scale2, shift2)

    return jnp.transpose(out_t, (3, 0, 2, 1))


def kernel(x_nchw, w1_oihw, g1, b1, w2_oihw, g2, b2):
    return _conv_block(x_nchw, w1_oihw, g1, b1, w2_oihw, g2, b2)
